# baseline (speedup 1.0000x reference)
import functools
import math

import jax
import jax.numpy as jnp
from jax import lax
from jax.experimental import pallas as pl
from jax.experimental.pallas import tpu as pltpu
from jax.experimental.pallas import tpu_sc as plsc

D_MODEL = 64
SCALE = math.sqrt(D_MODEL)

NC = 2
NS = 16
NW = NC * NS
LANES = 16
ROW_PAIR = 2 * D_MODEL

TBLK = 512
TBLK_T = 8
BBLK = 256


def _half_rows(V):
    return ((V + 1) // 2 + TBLK - 1) // TBLK * TBLK


def _pack_table(V):
    vh = _half_rows(V)
    n_blk = vh // TBLK

    def body(tl_ref, tr_ref, out_ref):
        e = (
            lax.broadcasted_iota(jnp.int32, (D_MODEL, D_MODEL), 0)
            == lax.broadcasted_iota(jnp.int32, (D_MODEL, D_MODEL), 1)
        ).astype(jnp.float32)
        atl = lax.dot_general(
            tl_ref[...], e, (((0,), (0,)), ((), ())),
            preferred_element_type=jnp.float32,
            precision=lax.Precision.HIGHEST,
        )
        atr = lax.dot_general(
            tr_ref[...], e, (((0,), (0,)), ((), ())),
            preferred_element_type=jnp.float32,
            precision=lax.Precision.HIGHEST,
        )
        out_ref[...] = jnp.concatenate([atl, atr], axis=1) * SCALE

    nb_half = n_blk
    return pl.pallas_call(
        body,
        grid=(n_blk,),
        in_specs=[
            pl.BlockSpec((D_MODEL, TBLK), lambda j: (0, j)),
            pl.BlockSpec((D_MODEL, TBLK), lambda j: (0, j + nb_half)),
        ],
        out_specs=pl.BlockSpec((TBLK, ROW_PAIR), lambda j: (j, 0)),
        out_shape=jax.ShapeDtypeStruct((vh, ROW_PAIR), jnp.float32),
    )


def _gather(seq, batch, vh):
    assert batch % BBLK == 0 and seq % TBLK_T == 0
    n_units = (seq // TBLK_T) * (batch // BBLK)

    mesh = plsc.VectorSubcoreMesh(core_axis_name="c", subcore_axis_name="s")

    @functools.partial(
        pl.kernel,
        out_type=jax.ShapeDtypeStruct((seq, D_MODEL, batch), jnp.float32),
        mesh=mesh,
        scratch_types=[
            pltpu.VMEM((TBLK_T, BBLK), jnp.int32),
            pltpu.VMEM((BBLK,), jnp.int32),
            pltpu.VMEM((BBLK,), jnp.int32),
            pltpu.VMEM((BBLK, ROW_PAIR), jnp.float32),
            pltpu.VMEM((D_MODEL, BBLK), jnp.float32),
            pltpu.SemaphoreType.DMA,
        ],
        compiler_params=pltpu.CompilerParams(needs_layout_passes=False),
    )
    def emb(xT_hbm, t2_hbm, out_hbm, idx_v, pv, ov, gbuf, obuf, sem):
        wid = lax.axis_index("s") * NC + lax.axis_index("c")
        n_my = (n_units - wid + NW - 1) // NW

        def unit_body(k, carry):
            uid = wid + k * NW
            t_blk = uid // (batch // BBLK)
            b0 = (uid % (batch // BBLK)) * BBLK
            t0 = t_blk * TBLK_T
            pltpu.sync_copy(
                xT_hbm.at[pl.ds(t0, TBLK_T), pl.ds(b0, BBLK)], idx_v
            )

            def t_body(tt, c):
                def prep(i, c2):
                    sl = pl.ds(i * LANES, LANES)
                    iv = idx_v[tt, sl]
                    m = ((vh - 1 - iv) >> 31) & 1
                    pv[sl] = iv - m * vh
                    ov[sl] = m * D_MODEL
                    return c2

                lax.fori_loop(0, BBLK // LANES, prep, 0)
                pltpu.async_copy(t2_hbm.at[pv], gbuf, sem).wait()

                def bv_body(bv, c3):
                    sl = pl.ds(bv * LANES, LANES)
                    offv = ov[sl]
                    rowv = lax.iota(jnp.int32, LANES) + bv * LANES
                    for d in range(D_MODEL):
                        val = plsc.load_gather(gbuf, [rowv, offv + d])
                        obuf[d, sl] = val
                    return c3

                lax.fori_loop(0, BBLK // LANES, bv_body, 0)
                pltpu.sync_copy(
                    obuf, out_hbm.at[t0 + tt, :, pl.ds(b0, BBLK)]
                )
                return c

            lax.fori_loop(0, TBLK_T, t_body, 0)
            return carry

        lax.fori_loop(0, n_my, unit_body, 0)

    return emb


def kernel(x, table):
    n_xrows, seq = x.shape
    V = table.shape[0]
    vh = _half_rows(V)
    tT = table.T
    t2 = _pack_table(V)(tT, tT)
    xT = x.T.astype(jnp.int32)
    out_phys = _gather(seq, n_xrows, vh)(xT, t2)
    return jnp.transpose(out_phys, (2, 0, 1))


# speedup vs baseline: 1.2785x; 1.2785x over previous
import functools
import math

import jax
import jax.numpy as jnp
from jax import lax
from jax.experimental import pallas as pl
from jax.experimental.pallas import tpu as pltpu
from jax.experimental.pallas import tpu_sc as plsc

D_MODEL = 64
SCALE = math.sqrt(D_MODEL)

NC = 2
NS = 16
NW = NC * NS
LANES = 16
ROW_PAIR = 2 * D_MODEL

TBLK = 2048
TBLK_T = 8
BBLK = 256


def _half_rows(V):
    return ((V + 1) // 2 + TBLK - 1) // TBLK * TBLK


def _pack_table(V):
    vh = _half_rows(V)
    n_blk = vh // TBLK

    def body(tl_ref, tr_ref, out_ref):
        e = (
            lax.broadcasted_iota(jnp.int32, (D_MODEL, D_MODEL), 0)
            == lax.broadcasted_iota(jnp.int32, (D_MODEL, D_MODEL), 1)
        ).astype(jnp.float32)
        atl = lax.dot_general(
            tl_ref[...], e, (((0,), (0,)), ((), ())),
            preferred_element_type=jnp.float32,
            precision=lax.Precision.HIGHEST,
        )
        atr = lax.dot_general(
            tr_ref[...], e, (((0,), (0,)), ((), ())),
            preferred_element_type=jnp.float32,
            precision=lax.Precision.HIGHEST,
        )
        out_ref[...] = jnp.concatenate([atl, atr], axis=1) * SCALE

    nb_half = n_blk
    last_blk = (V - 1) // TBLK
    return pl.pallas_call(
        body,
        grid=(n_blk,),
        in_specs=[
            pl.BlockSpec((D_MODEL, TBLK), lambda j: (0, j)),
            pl.BlockSpec(
                (D_MODEL, TBLK),
                lambda j: (0, jnp.minimum(j + nb_half, last_blk)),
            ),
        ],
        out_specs=pl.BlockSpec((TBLK, ROW_PAIR), lambda j: (j, 0)),
        out_shape=jax.ShapeDtypeStruct((vh, ROW_PAIR), jnp.float32),
    )


def _gather(seq, batch, vh):
    assert batch % BBLK == 0 and seq % TBLK_T == 0
    n_units = (seq // TBLK_T) * (batch // BBLK)

    mesh = plsc.VectorSubcoreMesh(core_axis_name="c", subcore_axis_name="s")

    @functools.partial(
        pl.kernel,
        out_type=jax.ShapeDtypeStruct((seq, D_MODEL, batch), jnp.float32),
        mesh=mesh,
        scratch_types=[
            pltpu.VMEM((TBLK_T, BBLK), jnp.int32),
            *[pltpu.VMEM((BBLK,), jnp.int32) for _ in range(TBLK_T)],
            pltpu.VMEM((TBLK_T, BBLK), jnp.int32),
            pltpu.VMEM((BBLK, ROW_PAIR), jnp.float32),
            pltpu.VMEM((BBLK, ROW_PAIR), jnp.float32),
            pltpu.VMEM((D_MODEL, BBLK), jnp.float32),
            pltpu.SemaphoreType.DMA,
        ],
        compiler_params=pltpu.CompilerParams(needs_layout_passes=False),
    )
    def emb(xT_hbm, t2_hbm, out_hbm, idx_v, *rest):
        (p0, p1, p2, p3, p4, p5, p6, p7, ov, gb0, gb1, obuf, sem) = rest
        pvs = (p0, p1, p2, p3, p4, p5, p6, p7)
        wid = lax.axis_index("s") * NC + lax.axis_index("c")
        n_my = (n_units - wid + NW - 1) // NW
        gbufs = (gb0, gb1)

        def unit_body(k, carry):
            uid = wid + k * NW
            t_blk = uid // (batch // BBLK)
            b0 = (uid % (batch // BBLK)) * BBLK
            t0 = t_blk * TBLK_T
            pltpu.sync_copy(
                xT_hbm.at[pl.ds(t0, TBLK_T), pl.ds(b0, BBLK)], idx_v
            )

            for tt_s in range(TBLK_T):
                def prep(i, c2, tt_s=tt_s):
                    sl = pl.ds(i * LANES, LANES)
                    iv = idx_v[tt_s, sl]
                    m = ((vh - 1 - iv) >> 31) & 1
                    pvs[tt_s][sl] = iv - m * vh
                    ov[tt_s, sl] = m * D_MODEL
                    return c2

                lax.fori_loop(0, BBLK // LANES, prep, 0)

            cops = [None] * TBLK_T
            cops[0] = pltpu.async_copy(t2_hbm.at[pvs[0]], gb0, sem)
            for tt in range(TBLK_T):
                gbuf = gbufs[tt % 2]
                if tt + 1 < TBLK_T:
                    cops[tt + 1] = pltpu.async_copy(
                        t2_hbm.at[pvs[tt + 1]], gbufs[(tt + 1) % 2], sem
                    )
                cops[tt].wait()

                def bv_body(bv, c3):
                    sl = pl.ds(bv * LANES, LANES)
                    offv = ov[tt, sl]
                    rowv = lax.iota(jnp.int32, LANES) + bv * LANES
                    for d in range(D_MODEL):
                        val = plsc.load_gather(gbuf, [rowv, offv + d])
                        obuf[d, sl] = val
                    return c3

                lax.fori_loop(0, BBLK // LANES, bv_body, 0)
                pltpu.sync_copy(
                    obuf, out_hbm.at[t0 + tt, :, pl.ds(b0, BBLK)]
                )
            return carry

        lax.fori_loop(0, n_my, unit_body, 0)

    return emb


def kernel(x, table):
    n_xrows, seq = x.shape
    V = table.shape[0]
    vh = _half_rows(V)
    tT = table.T
    t2 = _pack_table(V)(tT, tT)
    xT = x.T.astype(jnp.int32)
    out_phys = _gather(seq, n_xrows, vh)(xT, t2)
    return jnp.transpose(out_phys, (2, 0, 1))


# speedup vs baseline: 1.9871x; 1.5542x over previous
import functools
import math

import jax
import jax.numpy as jnp
from jax import lax
from jax.experimental import pallas as pl
from jax.experimental.pallas import tpu as pltpu
from jax.experimental.pallas import tpu_sc as plsc

D_MODEL = 64
SCALE = math.sqrt(D_MODEL)

NC = 2
NS = 16
NW = NC * NS
LANES = 16

CHUNK = 512


def _build(B):
    assert B % NW == 0
    b_per_w = B // NW
    assert b_per_w % (2 * CHUNK) == 0
    n_pairs = b_per_w // (2 * CHUNK)
    last = b_per_w // CHUNK - 1

    mesh = plsc.VectorSubcoreMesh(core_axis_name="c", subcore_axis_name="s")

    @functools.partial(
        pl.kernel,
        out_type=jax.ShapeDtypeStruct((B * D_MODEL,), jnp.float32),
        mesh=mesh,
        scratch_types=[
            pltpu.VMEM((b_per_w,), jnp.int32),
            pltpu.VMEM((CHUNK, D_MODEL), jnp.float32),
            pltpu.VMEM((CHUNK, D_MODEL), jnp.float32),
            pltpu.VMEM((CHUNK * D_MODEL,), jnp.float32),
            pltpu.SemaphoreType.DMA,
        ],
        compiler_params=pltpu.CompilerParams(use_tc_tiling_on_sc=False),
    )
    def emb(idx_hbm, table_hbm, out_hbm, idx_v, gb0, gb1, wbuf, sem):
        wid = lax.axis_index("s") * NC + lax.axis_index("c")
        base = wid * b_per_w
        pltpu.sync_copy(idx_hbm.at[pl.ds(base, b_per_w)], idx_v)

        def gather(g, buf):
            return pltpu.async_copy(
                table_hbm.at[idx_v.at[pl.ds(g * CHUNK, CHUNK)]], buf, sem
            )

        def scale_write(g, buf):
            def scale_row(i, c):
                for j in range(D_MODEL // LANES):
                    wbuf[pl.ds(i * D_MODEL + j * LANES, LANES)] = (
                        buf[i, pl.ds(j * LANES, LANES)] * SCALE
                    )
                return c

            lax.fori_loop(0, CHUNK, scale_row, 0)
            pltpu.sync_copy(
                wbuf,
                out_hbm.at[pl.ds((base + g * CHUNK) * D_MODEL, CHUNK * D_MODEL)],
            )

        gather(0, gb0).wait()

        def pair_body(go, carry):
            g0 = 2 * go
            c1 = gather(g0 + 1, gb1)
            scale_write(g0, gb0)
            c1.wait()
            g2 = jnp.minimum(g0 + 2, last)
            c2 = gather(g2, gb0)
            scale_write(g0 + 1, gb1)
            c2.wait()
            return carry

        lax.fori_loop(0, n_pairs, pair_body, 0)

    return emb


def kernel(x, table):
    orig_shape = x.shape
    xf = x.reshape(-1).astype(jnp.int32)
    out = _build(xf.shape[0])(xf, table)
    return out.reshape(orig_shape + (D_MODEL,))
